# baseline (speedup 1.0000x reference)
import functools

import jax
import jax.numpy as jnp
from jax import lax
from jax.experimental import pallas as pl
from jax.experimental.pallas import tpu as pltpu
from jax.experimental.pallas import tpu_sc as plsc

F32 = jnp.float32

_NC = 2
_NS = 16
_NW = _NC * _NS
_C = 80
_U = 80


def _sc_mesh():
    return plsc.VectorSubcoreMesh(core_axis_name="c", subcore_axis_name="s")


_SC_PARAMS = pltpu.CompilerParams(use_tc_tiling_on_sc=False)


def _zero_acc(acc, zbuf, s, d, nunits):
    def zinit(i, carry):
        for j in range(d // 16):
            zbuf[i, pl.ds(j * 16, 16)] = jnp.zeros((16,), F32)
        return carry

    lax.fori_loop(0, _U, zinit, 0)

    def zcopy(m, carry):
        t = s + m * _NS

        @pl.when(t < nunits)
        def _():
            pltpu.sync_copy(zbuf, acc.at[pl.ds(t * _U, _U)])

        return carry

    lax.fori_loop(0, (nunits + _NS - 1) // _NS, zcopy, 0)


def _copy_out(acc, out_h, c, s, nunits):
    def ocopy(m, carry):
        t = s + m * _NS

        @pl.when(t < nunits)
        def _():
            pltpu.sync_copy(acc.at[pl.ds(t * _U, _U)],
                            out_h.at[c, pl.ds(t * _U, _U)])

        return carry

    lax.fori_loop(0, (nunits + _NS - 1) // _NS, ocopy, 0)


def _deg_partials(edges4, n):
    _, nw, nch, c_ = edges4.shape
    nunits = n // _U

    @functools.partial(
        pl.kernel,
        out_type=jax.ShapeDtypeStruct((_NC, n, 16), F32),
        mesh=_sc_mesh(),
        compiler_params=_SC_PARAMS,
        scratch_types=[
            pltpu.VMEM_SHARED((n, 16), F32),
            pltpu.VMEM((nch, _C), jnp.int32),
            pltpu.VMEM((_C, 16), F32),
            pltpu.VMEM((_U, 16), F32),
            pltpu.SemaphoreType.DMA,
            pltpu.SemaphoreType.DMA,
        ],
    )
    def k(edges_h, out_h, acc, uidx, ones_v, zbuf, isem, ssem):
        c = lax.axis_index("c")
        s = lax.axis_index("s")
        w = c * _NS + s

        pltpu.async_copy(edges_h.at[0, w], uidx, isem)

        def init(i, carry):
            ones_v[i, :] = jnp.full((16,), 1.0, F32)
            return carry

        lax.fori_loop(0, _C, init, 0)
        _zero_acc(acc, zbuf, s, 16, nunits)
        plsc.subcore_barrier()
        pltpu.make_async_copy(edges_h.at[0, w], uidx, isem).wait()

        lag = 4

        def body(kk, carry):
            pltpu.async_copy(ones_v, acc.at[uidx.at[kk]], ssem, add=True)

            @pl.when(kk >= lag)
            def _():
                pltpu.make_async_copy(ones_v, acc.at[uidx.at[kk]], ssem).wait()

            return carry

        lax.fori_loop(0, nch, body, 0)
        for _ in range(min(lag, nch)):
            pltpu.make_async_copy(ones_v, acc.at[uidx.at[0]], ssem).wait()
        plsc.subcore_barrier()
        _copy_out(acc, out_h, c, s, nunits)

    return k(edges4)


def _spmm_partials(table, edges4):
    n, d = table.shape
    _, nw, nch, c_ = edges4.shape
    nunits = n // _U
    spmem_table = False
    nbuf = 3 if d >= 128 else 8
    ngrp = nch // nbuf
    ntail = nch - ngrp * nbuf

    scratch = [
        pltpu.VMEM_SHARED((n, d), F32),
        pltpu.VMEM((nch, _C), jnp.int32),
        pltpu.VMEM((nch, _C), jnp.int32),
        [pltpu.VMEM((_C, d), F32)] * nbuf,
        pltpu.SemaphoreType.DMA,
        [pltpu.SemaphoreType.DMA] * nbuf,
        [pltpu.SemaphoreType.DMA] * nbuf,
    ]
    if spmem_table:
        scratch.append(pltpu.VMEM_SHARED((n, d), F32))

    @functools.partial(
        pl.kernel,
        out_type=jax.ShapeDtypeStruct((_NC, n, d), F32),
        mesh=_sc_mesh(),
        compiler_params=_SC_PARAMS,
        scratch_types=scratch,
    )
    def k(table_h, edges_h, out_h, acc, vidx, uidx, rows,
          isem, gsem, ssem, *maybe_tsh):
        c = lax.axis_index("c")
        s = lax.axis_index("s")
        w = c * _NS + s

        pltpu.async_copy(edges_h.at[1, w], vidx, isem)
        pltpu.async_copy(edges_h.at[0, w], uidx, isem)
        _zero_acc(acc, rows[0], s, d, nunits)
        if spmem_table:
            tsh = maybe_tsh[0]

            def tcopy(m, carry):
                t = s + m * _NS

                @pl.when(t < nunits)
                def _():
                    pltpu.sync_copy(table_h.at[pl.ds(t * _U, _U)],
                                    tsh.at[pl.ds(t * _U, _U)])

                return carry

            lax.fori_loop(0, (nunits + _NS - 1) // _NS, tcopy, 0)
            src = tsh
        else:
            src = table_h
        pltpu.make_async_copy(edges_h.at[1, w], vidx, isem).wait()
        pltpu.make_async_copy(edges_h.at[0, w], uidx, isem).wait()
        if not spmem_table:
            for b in range(nbuf):
                pltpu.async_copy(src.at[vidx.at[b]], rows[b], gsem[b])
        plsc.subcore_barrier()
        if spmem_table:
            for b in range(nbuf):
                pltpu.async_copy(src.at[vidx.at[b]], rows[b], gsem[b])

        def stage(kk, b):
            pltpu.make_async_copy(src.at[vidx.at[kk]], rows[b],
                                  gsem[b]).wait()
            pltpu.async_copy(rows[b], acc.at[uidx.at[kk]], ssem[b], add=True)
            pltpu.make_async_copy(rows[b], acc.at[uidx.at[kk]],
                                  ssem[b]).wait()

            @pl.when(kk + nbuf < nch)
            def _():
                pltpu.async_copy(src.at[vidx.at[kk + nbuf]], rows[b],
                                 gsem[b])

        def body(j, carry):
            for b in range(nbuf):
                stage(nbuf * j + b, b)
            return carry

        lax.fori_loop(0, ngrp, body, 0)
        for t in range(ntail):
            kk = ngrp * nbuf + t
            pltpu.make_async_copy(src.at[vidx.at[kk]], rows[t],
                                  gsem[t]).wait()
            pltpu.sync_copy(rows[t], acc.at[uidx.at[kk]], add=True)
        plsc.subcore_barrier()
        _copy_out(acc, out_h, c, s, nunits)

    return k(table, edges4)


_BN = 1000


def _xw(x, w1):
    n, din = x.shape
    dh = w1.shape[1]

    def body(x_ref, w_ref, xw_ref):
        xw_ref[...] = jnp.dot(x_ref[...], w_ref[...],
                              preferred_element_type=F32)

    return pl.pallas_call(
        body,
        grid=(n // _BN,),
        in_specs=[
            pl.BlockSpec((_BN, din), lambda i: (i, 0)),
            pl.BlockSpec((din, dh), lambda i: (0, 0)),
        ],
        out_specs=pl.BlockSpec((_BN, dh), lambda i: (i, 0)),
        out_shape=jax.ShapeDtypeStruct((n, dh), F32),
    )(x, w1)


def _pre(xw, degp):
    n, dh = xw.shape

    def body(xw_ref, dp_ref, xws_ref, isd_ref):
        deg = dp_ref[0] + dp_ref[1]
        isd = jnp.where(deg > 0, lax.rsqrt(deg), 0.0)
        isd_col = isd[:, 0:1]
        xws_ref[...] = xw_ref[...] * isd_col
        isd_ref[...] = isd_col

    return pl.pallas_call(
        body,
        grid=(n // _BN,),
        in_specs=[
            pl.BlockSpec((_BN, dh), lambda i: (i, 0)),
            pl.BlockSpec((2, _BN, 16), lambda i: (0, i, 0)),
        ],
        out_specs=[
            pl.BlockSpec((_BN, dh), lambda i: (i, 0)),
            pl.BlockSpec((_BN, 1), lambda i: (i, 0)),
        ],
        out_shape=[
            jax.ShapeDtypeStruct((n, dh), F32),
            jax.ShapeDtypeStruct((n, 1), F32),
        ],
    )(xw, degp)


def _mid(accp, isd, w2):
    _, n, dh = accp.shape
    dout = w2.shape[1]

    def body(a_ref, isd_ref, w_ref, out_ref):
        isd_col = isd_ref[...]
        h = jnp.maximum((a_ref[0] + a_ref[1]) * isd_col, 0.0)
        out_ref[...] = jnp.dot(h, w_ref[...],
                               preferred_element_type=F32) * isd_col

    return pl.pallas_call(
        body,
        grid=(n // _BN,),
        in_specs=[
            pl.BlockSpec((2, _BN, dh), lambda i: (0, i, 0)),
            pl.BlockSpec((_BN, 1), lambda i: (i, 0)),
            pl.BlockSpec((dh, dout), lambda i: (0, 0)),
        ],
        out_specs=pl.BlockSpec((_BN, dout), lambda i: (i, 0)),
        out_shape=jax.ShapeDtypeStruct((n, dout), F32),
    )(accp, isd, w2)


def _post(p, isd):
    _, n, dout = p.shape

    def body(p_ref, isd_ref, out_ref):
        out_ref[...] = (p_ref[0] + p_ref[1]) * isd_ref[...]

    return pl.pallas_call(
        body,
        grid=(n // _BN,),
        in_specs=[
            pl.BlockSpec((2, _BN, dout), lambda i: (0, i, 0)),
            pl.BlockSpec((_BN, 1), lambda i: (i, 0)),
        ],
        out_specs=pl.BlockSpec((_BN, dout), lambda i: (i, 0)),
        out_shape=jax.ShapeDtypeStruct((n, dout), F32),
    )(p, isd)


def kernel(x, edge_index, W1, W2):
    n = x.shape[0]
    e = edge_index.shape[1]
    epw = e // _NW
    edges4 = edge_index.reshape(2, _NW, epw // _C, _C)
    xw = _xw(x, W1)
    degp = _deg_partials(edges4, n)
    xws, isd = _pre(xw, degp)
    accp = _spmm_partials(xws, edges4)
    hws = _mid(accp, isd, W2)
    p2 = _spmm_partials(hws, edges4)
    return _post(p2, isd)


# speedup vs baseline: 1.0266x; 1.0266x over previous
import functools

import jax
import jax.numpy as jnp
from jax import lax
from jax.experimental import pallas as pl
from jax.experimental.pallas import tpu as pltpu
from jax.experimental.pallas import tpu_sc as plsc

F32 = jnp.float32

_NC = 2
_NS = 16
_NW = _NC * _NS
_C = 80
_U = 80


def _sc_mesh():
    return plsc.VectorSubcoreMesh(core_axis_name="c", subcore_axis_name="s")


_SC_PARAMS = pltpu.CompilerParams(use_tc_tiling_on_sc=False)


def _zero_acc(acc, zbuf, s, d, nunits):
    def zinit(i, carry):
        for j in range(d // 16):
            zbuf[i, pl.ds(j * 16, 16)] = jnp.zeros((16,), F32)
        return carry

    lax.fori_loop(0, _U, zinit, 0)

    def zcopy(m, carry):
        t = s + m * _NS

        @pl.when(t < nunits)
        def _():
            pltpu.sync_copy(zbuf, acc.at[pl.ds(t * _U, _U)])

        return carry

    lax.fori_loop(0, (nunits + _NS - 1) // _NS, zcopy, 0)


def _copy_out(acc, out_h, c, s, nunits):
    def ocopy(m, carry):
        t = s + m * _NS

        @pl.when(t < nunits)
        def _():
            pltpu.sync_copy(acc.at[pl.ds(t * _U, _U)],
                            out_h.at[c, pl.ds(t * _U, _U)])

        return carry

    lax.fori_loop(0, (nunits + _NS - 1) // _NS, ocopy, 0)


def _deg_partials(edges4, n):
    _, nw, nch, c_ = edges4.shape
    nunits = n // _U

    @functools.partial(
        pl.kernel,
        out_type=jax.ShapeDtypeStruct((_NC, n, 16), F32),
        mesh=_sc_mesh(),
        compiler_params=_SC_PARAMS,
        scratch_types=[
            pltpu.VMEM_SHARED((n, 16), F32),
            pltpu.VMEM((nch, _C), jnp.int32),
            pltpu.VMEM((_C, 16), F32),
            pltpu.VMEM((_U, 16), F32),
            pltpu.SemaphoreType.DMA,
            pltpu.SemaphoreType.DMA,
        ],
    )
    def k(edges_h, out_h, acc, uidx, ones_v, zbuf, isem, ssem):
        c = lax.axis_index("c")
        s = lax.axis_index("s")
        w = c * _NS + s

        pltpu.async_copy(edges_h.at[0, w], uidx, isem)

        def init(i, carry):
            ones_v[i, :] = jnp.full((16,), 1.0, F32)
            return carry

        lax.fori_loop(0, _C, init, 0)
        _zero_acc(acc, zbuf, s, 16, nunits)
        plsc.subcore_barrier()
        pltpu.make_async_copy(edges_h.at[0, w], uidx, isem).wait()

        lag = 4

        def body(kk, carry):
            pltpu.async_copy(ones_v, acc.at[uidx.at[kk]], ssem, add=True)

            @pl.when(kk >= lag)
            def _():
                pltpu.make_async_copy(ones_v, acc.at[uidx.at[kk]], ssem).wait()

            return carry

        lax.fori_loop(0, nch, body, 0)
        for _ in range(min(lag, nch)):
            pltpu.make_async_copy(ones_v, acc.at[uidx.at[0]], ssem).wait()
        plsc.subcore_barrier()
        _copy_out(acc, out_h, c, s, nunits)

    return k(edges4)


def _spmm_partials(table, edges4):
    n, d = table.shape
    _, nw, nch, c_ = edges4.shape
    nunits = n // _U
    spmem_table = False
    nbuf = 3 if d >= 128 else 8
    ngrp = nch // nbuf
    ntail = nch - ngrp * nbuf

    scratch = [
        pltpu.VMEM_SHARED((n, d), F32),
        pltpu.VMEM((nch, _C), jnp.int32),
        pltpu.VMEM((nch, _C), jnp.int32),
        [pltpu.VMEM((_C, d), F32)] * nbuf,
        pltpu.SemaphoreType.DMA,
        [pltpu.SemaphoreType.DMA] * nbuf,
        [pltpu.SemaphoreType.DMA] * nbuf,
    ]
    if spmem_table:
        scratch.append(pltpu.VMEM_SHARED((n, d), F32))

    @functools.partial(
        pl.kernel,
        out_type=jax.ShapeDtypeStruct((_NC, n, d), F32),
        mesh=_sc_mesh(),
        compiler_params=_SC_PARAMS,
        scratch_types=scratch,
    )
    def k(table_h, edges_h, out_h, acc, vidx, uidx, rows,
          isem, gsem, ssem, *maybe_tsh):
        c = lax.axis_index("c")
        s = lax.axis_index("s")
        w = c * _NS + s

        pltpu.async_copy(edges_h.at[1, w], vidx, isem)
        pltpu.async_copy(edges_h.at[0, w], uidx, isem)
        _zero_acc(acc, rows[0], s, d, nunits)
        if spmem_table:
            tsh = maybe_tsh[0]

            def tcopy(m, carry):
                t = s + m * _NS

                @pl.when(t < nunits)
                def _():
                    pltpu.sync_copy(table_h.at[pl.ds(t * _U, _U)],
                                    tsh.at[pl.ds(t * _U, _U)])

                return carry

            lax.fori_loop(0, (nunits + _NS - 1) // _NS, tcopy, 0)
            src = tsh
        else:
            src = table_h
        pltpu.make_async_copy(edges_h.at[1, w], vidx, isem).wait()
        pltpu.make_async_copy(edges_h.at[0, w], uidx, isem).wait()
        if not spmem_table:
            for b in range(nbuf):
                pltpu.async_copy(src.at[vidx.at[b]], rows[b], gsem[b])
        plsc.subcore_barrier()
        if spmem_table:
            for b in range(nbuf):
                pltpu.async_copy(src.at[vidx.at[b]], rows[b], gsem[b])

        def stage(kk, b):
            pltpu.make_async_copy(src.at[vidx.at[kk]], rows[b],
                                  gsem[b]).wait()
            pltpu.async_copy(rows[b], acc.at[uidx.at[kk]], ssem[b], add=True)
            pltpu.make_async_copy(rows[b], acc.at[uidx.at[kk]],
                                  ssem[b]).wait()

            @pl.when(kk + nbuf < nch)
            def _():
                pltpu.async_copy(src.at[vidx.at[kk + nbuf]], rows[b],
                                 gsem[b])

        def body(j, carry):
            for b in range(nbuf):
                stage(nbuf * j + b, b)
            return carry

        lax.fori_loop(0, ngrp, body, 0)
        for t in range(ntail):
            kk = ngrp * nbuf + t
            pltpu.make_async_copy(src.at[vidx.at[kk]], rows[t],
                                  gsem[t]).wait()
            pltpu.sync_copy(rows[t], acc.at[uidx.at[kk]], add=True)
        plsc.subcore_barrier()
        _copy_out(acc, out_h, c, s, nunits)

    return k(table, edges4)


_BN = 2000


def _xw(x, w1):
    n, din = x.shape
    dh = w1.shape[1]

    def body(x_ref, w_ref, xw_ref):
        xw_ref[...] = jnp.dot(x_ref[...], w_ref[...],
                              preferred_element_type=F32)

    return pl.pallas_call(
        body,
        grid=(n // _BN,),
        in_specs=[
            pl.BlockSpec((_BN, din), lambda i: (i, 0)),
            pl.BlockSpec((din, dh), lambda i: (0, 0)),
        ],
        out_specs=pl.BlockSpec((_BN, dh), lambda i: (i, 0)),
        out_shape=jax.ShapeDtypeStruct((n, dh), F32),
    )(x, w1)


def _pre(xw, degp):
    n, dh = xw.shape

    def body(xw_ref, dp_ref, xws_ref, isd_ref):
        deg = dp_ref[0] + dp_ref[1]
        isd = jnp.where(deg > 0, lax.rsqrt(deg), 0.0)
        isd_col = isd[:, 0:1]
        xws_ref[...] = xw_ref[...] * isd_col
        isd_ref[...] = isd_col

    return pl.pallas_call(
        body,
        grid=(n // _BN,),
        in_specs=[
            pl.BlockSpec((_BN, dh), lambda i: (i, 0)),
            pl.BlockSpec((2, _BN, 16), lambda i: (0, i, 0)),
        ],
        out_specs=[
            pl.BlockSpec((_BN, dh), lambda i: (i, 0)),
            pl.BlockSpec((_BN, 1), lambda i: (i, 0)),
        ],
        out_shape=[
            jax.ShapeDtypeStruct((n, dh), F32),
            jax.ShapeDtypeStruct((n, 1), F32),
        ],
    )(xw, degp)


def _mid(accp, isd, w2):
    _, n, dh = accp.shape
    dout = w2.shape[1]

    def body(a_ref, isd_ref, w_ref, out_ref):
        isd_col = isd_ref[...]
        h = jnp.maximum((a_ref[0] + a_ref[1]) * isd_col, 0.0)
        out_ref[...] = jnp.dot(h, w_ref[...],
                               preferred_element_type=F32) * isd_col

    return pl.pallas_call(
        body,
        grid=(n // _BN,),
        in_specs=[
            pl.BlockSpec((2, _BN, dh), lambda i: (0, i, 0)),
            pl.BlockSpec((_BN, 1), lambda i: (i, 0)),
            pl.BlockSpec((dh, dout), lambda i: (0, 0)),
        ],
        out_specs=pl.BlockSpec((_BN, dout), lambda i: (i, 0)),
        out_shape=jax.ShapeDtypeStruct((n, dout), F32),
    )(accp, isd, w2)


def _post(p, isd):
    _, n, dout = p.shape

    def body(p_ref, isd_ref, out_ref):
        out_ref[...] = (p_ref[0] + p_ref[1]) * isd_ref[...]

    return pl.pallas_call(
        body,
        grid=(n // _BN,),
        in_specs=[
            pl.BlockSpec((2, _BN, dout), lambda i: (0, i, 0)),
            pl.BlockSpec((_BN, 1), lambda i: (i, 0)),
        ],
        out_specs=pl.BlockSpec((_BN, dout), lambda i: (i, 0)),
        out_shape=jax.ShapeDtypeStruct((n, dout), F32),
    )(p, isd)


def kernel(x, edge_index, W1, W2):
    n = x.shape[0]
    e = edge_index.shape[1]
    epw = e // _NW
    edges4 = edge_index.reshape(2, _NW, epw // _C, _C)
    xw = _xw(x, W1)
    degp = _deg_partials(edges4, n)
    xws, isd = _pre(xw, degp)
    accp = _spmm_partials(xws, edges4)
    hws = _mid(accp, isd, W2)
    p2 = _spmm_partials(hws, edges4)
    return _post(p2, isd)


# speedup vs baseline: 1.0495x; 1.0222x over previous
import functools

import jax
import jax.numpy as jnp
from jax import lax
from jax.experimental import pallas as pl
from jax.experimental.pallas import tpu as pltpu
from jax.experimental.pallas import tpu_sc as plsc

F32 = jnp.float32

_NC = 2
_NS = 16
_NW = _NC * _NS
_C = 80
_U = 80


def _sc_mesh():
    return plsc.VectorSubcoreMesh(core_axis_name="c", subcore_axis_name="s")


_SC_PARAMS = pltpu.CompilerParams(use_tc_tiling_on_sc=False)


def _zero_acc(acc, zbuf, s, d, nunits):
    def zinit(i, carry):
        for j in range(d // 16):
            zbuf[i, pl.ds(j * 16, 16)] = jnp.zeros((16,), F32)
        return carry

    lax.fori_loop(0, _U, zinit, 0)

    def zcopy(m, carry):
        t = s + m * _NS

        @pl.when(t < nunits)
        def _():
            pltpu.sync_copy(zbuf, acc.at[pl.ds(t * _U, _U)])

        return carry

    lax.fori_loop(0, (nunits + _NS - 1) // _NS, zcopy, 0)


def _copy_out(acc, out_h, c, s, nunits, osem):
    mtrips = (nunits + _NS - 1) // _NS

    def ocopy(m, carry):
        t = s + m * _NS

        @pl.when(t < nunits)
        def _():
            pltpu.async_copy(acc.at[pl.ds(t * _U, _U)],
                             out_h.at[c, pl.ds(t * _U, _U)], osem)

        return carry

    lax.fori_loop(0, mtrips, ocopy, 0)

    def odrain(m, carry):
        t = s + m * _NS

        @pl.when(t < nunits)
        def _():
            pltpu.make_async_copy(acc.at[pl.ds(t * _U, _U)],
                                  out_h.at[c, pl.ds(t * _U, _U)], osem).wait()

        return carry

    lax.fori_loop(0, mtrips, odrain, 0)


def _deg_partials(edges4, n):
    _, nw, nch, c_ = edges4.shape
    nunits = n // _U

    @functools.partial(
        pl.kernel,
        out_type=jax.ShapeDtypeStruct((_NC, n, 16), F32),
        mesh=_sc_mesh(),
        compiler_params=_SC_PARAMS,
        scratch_types=[
            pltpu.VMEM_SHARED((n, 16), F32),
            pltpu.VMEM((nch, _C), jnp.int32),
            pltpu.VMEM((_C, 16), F32),
            pltpu.VMEM((_U, 16), F32),
            pltpu.SemaphoreType.DMA,
            pltpu.SemaphoreType.DMA,
        ],
    )
    def k(edges_h, out_h, acc, uidx, ones_v, zbuf, isem, ssem):
        c = lax.axis_index("c")
        s = lax.axis_index("s")
        w = c * _NS + s

        pltpu.async_copy(edges_h.at[0, w], uidx, isem)

        def init(i, carry):
            ones_v[i, :] = jnp.full((16,), 1.0, F32)
            return carry

        lax.fori_loop(0, _C, init, 0)
        _zero_acc(acc, zbuf, s, 16, nunits)
        plsc.subcore_barrier()
        pltpu.make_async_copy(edges_h.at[0, w], uidx, isem).wait()

        lag = 4

        def body(kk, carry):
            pltpu.async_copy(ones_v, acc.at[uidx.at[kk]], ssem, add=True)

            @pl.when(kk >= lag)
            def _():
                pltpu.make_async_copy(ones_v, acc.at[uidx.at[kk]], ssem).wait()

            return carry

        lax.fori_loop(0, nch, body, 0)
        for _ in range(min(lag, nch)):
            pltpu.make_async_copy(ones_v, acc.at[uidx.at[0]], ssem).wait()
        plsc.subcore_barrier()
        _copy_out(acc, out_h, c, s, nunits, isem)

    return k(edges4)


def _spmm_partials(table, edges4):
    n, d = table.shape
    _, nw, nch, c_ = edges4.shape
    nunits = n // _U
    spmem_table = False
    nbuf = 3 if d >= 128 else 8
    ngrp = nch // nbuf
    ntail = nch - ngrp * nbuf

    scratch = [
        pltpu.VMEM_SHARED((n, d), F32),
        pltpu.VMEM((nch, _C), jnp.int32),
        pltpu.VMEM((nch, _C), jnp.int32),
        [pltpu.VMEM((_C, d), F32)] * nbuf,
        pltpu.SemaphoreType.DMA,
        [pltpu.SemaphoreType.DMA] * nbuf,
        [pltpu.SemaphoreType.DMA] * nbuf,
    ]
    if spmem_table:
        scratch.append(pltpu.VMEM_SHARED((n, d), F32))

    @functools.partial(
        pl.kernel,
        out_type=jax.ShapeDtypeStruct((_NC, n, d), F32),
        mesh=_sc_mesh(),
        compiler_params=_SC_PARAMS,
        scratch_types=scratch,
    )
    def k(table_h, edges_h, out_h, acc, vidx, uidx, rows,
          isem, gsem, ssem, *maybe_tsh):
        c = lax.axis_index("c")
        s = lax.axis_index("s")
        w = c * _NS + s

        pltpu.async_copy(edges_h.at[1, w], vidx, isem)
        pltpu.async_copy(edges_h.at[0, w], uidx, isem)
        _zero_acc(acc, rows[0], s, d, nunits)
        if spmem_table:
            tsh = maybe_tsh[0]

            def tcopy(m, carry):
                t = s + m * _NS

                @pl.when(t < nunits)
                def _():
                    pltpu.sync_copy(table_h.at[pl.ds(t * _U, _U)],
                                    tsh.at[pl.ds(t * _U, _U)])

                return carry

            lax.fori_loop(0, (nunits + _NS - 1) // _NS, tcopy, 0)
            src = tsh
        else:
            src = table_h
        pltpu.make_async_copy(edges_h.at[1, w], vidx, isem).wait()
        pltpu.make_async_copy(edges_h.at[0, w], uidx, isem).wait()
        if not spmem_table:
            for b in range(nbuf):
                pltpu.async_copy(src.at[vidx.at[b]], rows[b], gsem[b])
        plsc.subcore_barrier()
        if spmem_table:
            for b in range(nbuf):
                pltpu.async_copy(src.at[vidx.at[b]], rows[b], gsem[b])

        def stage(kk, b):
            pltpu.make_async_copy(src.at[vidx.at[kk]], rows[b],
                                  gsem[b]).wait()
            pltpu.async_copy(rows[b], acc.at[uidx.at[kk]], ssem[b], add=True)
            pltpu.make_async_copy(rows[b], acc.at[uidx.at[kk]],
                                  ssem[b]).wait()

            @pl.when(kk + nbuf < nch)
            def _():
                pltpu.async_copy(src.at[vidx.at[kk + nbuf]], rows[b],
                                 gsem[b])

        def body(j, carry):
            for b in range(nbuf):
                stage(nbuf * j + b, b)
            return carry

        lax.fori_loop(0, ngrp, body, 0)
        for t in range(ntail):
            kk = ngrp * nbuf + t
            pltpu.make_async_copy(src.at[vidx.at[kk]], rows[t],
                                  gsem[t]).wait()
            pltpu.sync_copy(rows[t], acc.at[uidx.at[kk]], add=True)
        plsc.subcore_barrier()
        _copy_out(acc, out_h, c, s, nunits, isem)

    return k(table, edges4)


_BN = 2000


def _xw(x, w1):
    n, din = x.shape
    dh = w1.shape[1]

    def body(x_ref, w_ref, xw_ref):
        xw_ref[...] = jnp.dot(x_ref[...], w_ref[...],
                              preferred_element_type=F32)

    return pl.pallas_call(
        body,
        grid=(n // _BN,),
        in_specs=[
            pl.BlockSpec((_BN, din), lambda i: (i, 0)),
            pl.BlockSpec((din, dh), lambda i: (0, 0)),
        ],
        out_specs=pl.BlockSpec((_BN, dh), lambda i: (i, 0)),
        out_shape=jax.ShapeDtypeStruct((n, dh), F32),
    )(x, w1)


def _pre(xw, degp):
    n, dh = xw.shape

    def body(xw_ref, dp_ref, xws_ref, isd_ref):
        deg = dp_ref[0] + dp_ref[1]
        isd = jnp.where(deg > 0, lax.rsqrt(deg), 0.0)
        isd_col = isd[:, 0:1]
        xws_ref[...] = xw_ref[...] * isd_col
        isd_ref[...] = isd_col

    return pl.pallas_call(
        body,
        grid=(n // _BN,),
        in_specs=[
            pl.BlockSpec((_BN, dh), lambda i: (i, 0)),
            pl.BlockSpec((2, _BN, 16), lambda i: (0, i, 0)),
        ],
        out_specs=[
            pl.BlockSpec((_BN, dh), lambda i: (i, 0)),
            pl.BlockSpec((_BN, 1), lambda i: (i, 0)),
        ],
        out_shape=[
            jax.ShapeDtypeStruct((n, dh), F32),
            jax.ShapeDtypeStruct((n, 1), F32),
        ],
    )(xw, degp)


def _mid(accp, isd, w2):
    _, n, dh = accp.shape
    dout = w2.shape[1]

    def body(a_ref, isd_ref, w_ref, out_ref):
        isd_col = isd_ref[...]
        h = jnp.maximum((a_ref[0] + a_ref[1]) * isd_col, 0.0)
        out_ref[...] = jnp.dot(h, w_ref[...],
                               preferred_element_type=F32) * isd_col

    return pl.pallas_call(
        body,
        grid=(n // _BN,),
        in_specs=[
            pl.BlockSpec((2, _BN, dh), lambda i: (0, i, 0)),
            pl.BlockSpec((_BN, 1), lambda i: (i, 0)),
            pl.BlockSpec((dh, dout), lambda i: (0, 0)),
        ],
        out_specs=pl.BlockSpec((_BN, dout), lambda i: (i, 0)),
        out_shape=jax.ShapeDtypeStruct((n, dout), F32),
    )(accp, isd, w2)


def _post(p, isd):
    _, n, dout = p.shape

    def body(p_ref, isd_ref, out_ref):
        out_ref[...] = (p_ref[0] + p_ref[1]) * isd_ref[...]

    return pl.pallas_call(
        body,
        grid=(n // _BN,),
        in_specs=[
            pl.BlockSpec((2, _BN, dout), lambda i: (0, i, 0)),
            pl.BlockSpec((_BN, 1), lambda i: (i, 0)),
        ],
        out_specs=pl.BlockSpec((_BN, dout), lambda i: (i, 0)),
        out_shape=jax.ShapeDtypeStruct((n, dout), F32),
    )(p, isd)


def kernel(x, edge_index, W1, W2):
    n = x.shape[0]
    e = edge_index.shape[1]
    epw = e // _NW
    edges4 = edge_index.reshape(2, _NW, epw // _C, _C)
    xw = _xw(x, W1)
    degp = _deg_partials(edges4, n)
    xws, isd = _pre(xw, degp)
    accp = _spmm_partials(xws, edges4)
    hws = _mid(accp, isd, W2)
    p2 = _spmm_partials(hws, edges4)
    return _post(p2, isd)


# speedup vs baseline: 1.0520x; 1.0024x over previous
import functools

import jax
import jax.numpy as jnp
from jax import lax
from jax.experimental import pallas as pl
from jax.experimental.pallas import tpu as pltpu
from jax.experimental.pallas import tpu_sc as plsc

F32 = jnp.float32

_NC = 2
_NS = 16
_NW = _NC * _NS
_C = 80
_U = 80


def _sc_mesh():
    return plsc.VectorSubcoreMesh(core_axis_name="c", subcore_axis_name="s")


_SC_PARAMS = pltpu.CompilerParams(use_tc_tiling_on_sc=False)


def _zero_acc(acc, zbuf, s, d, nunits, zsem):
    mtrips = (nunits + _NS - 1) // _NS

    def zinit(i, carry):
        for j in range(d // 16):
            zbuf[i, pl.ds(j * 16, 16)] = jnp.zeros((16,), F32)
        return carry

    lax.fori_loop(0, _U, zinit, 0)

    def zcopy(m, carry):
        t = s + m * _NS

        @pl.when(t < nunits)
        def _():
            pltpu.async_copy(zbuf, acc.at[pl.ds(t * _U, _U)], zsem)

        return carry

    lax.fori_loop(0, mtrips, zcopy, 0)

    def zdrain(m, carry):
        t = s + m * _NS

        @pl.when(t < nunits)
        def _():
            pltpu.make_async_copy(zbuf, acc.at[pl.ds(t * _U, _U)],
                                  zsem).wait()

        return carry

    lax.fori_loop(0, mtrips, zdrain, 0)


def _copy_out(acc, out_h, c, s, nunits, osem):
    mtrips = (nunits + _NS - 1) // _NS

    def ocopy(m, carry):
        t = s + m * _NS

        @pl.when(t < nunits)
        def _():
            pltpu.async_copy(acc.at[pl.ds(t * _U, _U)],
                             out_h.at[c, pl.ds(t * _U, _U)], osem)

        return carry

    lax.fori_loop(0, mtrips, ocopy, 0)

    def odrain(m, carry):
        t = s + m * _NS

        @pl.when(t < nunits)
        def _():
            pltpu.make_async_copy(acc.at[pl.ds(t * _U, _U)],
                                  out_h.at[c, pl.ds(t * _U, _U)], osem).wait()

        return carry

    lax.fori_loop(0, mtrips, odrain, 0)


def _deg_partials(edges4, n):
    _, nw, nch, c_ = edges4.shape
    nunits = n // _U

    @functools.partial(
        pl.kernel,
        out_type=jax.ShapeDtypeStruct((_NC, n, 16), F32),
        mesh=_sc_mesh(),
        compiler_params=_SC_PARAMS,
        scratch_types=[
            pltpu.VMEM_SHARED((n, 16), F32),
            pltpu.VMEM((nch, _C), jnp.int32),
            pltpu.VMEM((_C, 16), F32),
            pltpu.VMEM((_U, 16), F32),
            pltpu.SemaphoreType.DMA,
            pltpu.SemaphoreType.DMA,
            pltpu.SemaphoreType.DMA,
        ],
    )
    def k(edges_h, out_h, acc, uidx, ones_v, zbuf, isem, ssem, zsem):
        c = lax.axis_index("c")
        s = lax.axis_index("s")
        w = c * _NS + s

        pltpu.async_copy(edges_h.at[0, w], uidx, isem)

        def init(i, carry):
            ones_v[i, :] = jnp.full((16,), 1.0, F32)
            return carry

        lax.fori_loop(0, _C, init, 0)
        _zero_acc(acc, zbuf, s, 16, nunits, zsem)
        plsc.subcore_barrier()
        pltpu.make_async_copy(edges_h.at[0, w], uidx, isem).wait()

        lag = 4

        def body(kk, carry):
            pltpu.async_copy(ones_v, acc.at[uidx.at[kk]], ssem, add=True)

            @pl.when(kk >= lag)
            def _():
                pltpu.make_async_copy(ones_v, acc.at[uidx.at[kk]], ssem).wait()

            return carry

        lax.fori_loop(0, nch, body, 0)
        for _ in range(min(lag, nch)):
            pltpu.make_async_copy(ones_v, acc.at[uidx.at[0]], ssem).wait()
        plsc.subcore_barrier()
        _copy_out(acc, out_h, c, s, nunits, isem)

    return k(edges4)


def _spmm_partials(table, edges4):
    n, d = table.shape
    _, nw, nch, c_ = edges4.shape
    nunits = n // _U
    spmem_table = False
    nbuf = 3 if d >= 128 else 8
    ngrp = nch // nbuf
    ntail = nch - ngrp * nbuf

    scratch = [
        pltpu.VMEM_SHARED((n, d), F32),
        pltpu.VMEM((nch, _C), jnp.int32),
        pltpu.VMEM((nch, _C), jnp.int32),
        [pltpu.VMEM((_C, d), F32)] * nbuf,
        pltpu.SemaphoreType.DMA,
        [pltpu.SemaphoreType.DMA] * nbuf,
        [pltpu.SemaphoreType.DMA] * nbuf,
        pltpu.SemaphoreType.DMA,
    ]
    if spmem_table:
        scratch.append(pltpu.VMEM_SHARED((n, d), F32))

    @functools.partial(
        pl.kernel,
        out_type=jax.ShapeDtypeStruct((_NC, n, d), F32),
        mesh=_sc_mesh(),
        compiler_params=_SC_PARAMS,
        scratch_types=scratch,
    )
    def k(table_h, edges_h, out_h, acc, vidx, uidx, rows,
          isem, gsem, ssem, zsem, *maybe_tsh):
        c = lax.axis_index("c")
        s = lax.axis_index("s")
        w = c * _NS + s

        pltpu.async_copy(edges_h.at[1, w], vidx, isem)
        pltpu.async_copy(edges_h.at[0, w], uidx, isem)
        _zero_acc(acc, rows[0], s, d, nunits, zsem)
        if spmem_table:
            tsh = maybe_tsh[0]

            def tcopy(m, carry):
                t = s + m * _NS

                @pl.when(t < nunits)
                def _():
                    pltpu.sync_copy(table_h.at[pl.ds(t * _U, _U)],
                                    tsh.at[pl.ds(t * _U, _U)])

                return carry

            lax.fori_loop(0, (nunits + _NS - 1) // _NS, tcopy, 0)
            src = tsh
        else:
            src = table_h
        pltpu.make_async_copy(edges_h.at[1, w], vidx, isem).wait()
        pltpu.make_async_copy(edges_h.at[0, w], uidx, isem).wait()
        if not spmem_table:
            for b in range(nbuf):
                pltpu.async_copy(src.at[vidx.at[b]], rows[b], gsem[b])
        plsc.subcore_barrier()
        if spmem_table:
            for b in range(nbuf):
                pltpu.async_copy(src.at[vidx.at[b]], rows[b], gsem[b])

        def stage(kk, b):
            pltpu.make_async_copy(src.at[vidx.at[kk]], rows[b],
                                  gsem[b]).wait()
            pltpu.async_copy(rows[b], acc.at[uidx.at[kk]], ssem[b], add=True)
            pltpu.make_async_copy(rows[b], acc.at[uidx.at[kk]],
                                  ssem[b]).wait()

            @pl.when(kk + nbuf < nch)
            def _():
                pltpu.async_copy(src.at[vidx.at[kk + nbuf]], rows[b],
                                 gsem[b])

        def body(j, carry):
            for b in range(nbuf):
                stage(nbuf * j + b, b)
            return carry

        lax.fori_loop(0, ngrp, body, 0)
        for t in range(ntail):
            kk = ngrp * nbuf + t
            pltpu.make_async_copy(src.at[vidx.at[kk]], rows[t],
                                  gsem[t]).wait()
            pltpu.sync_copy(rows[t], acc.at[uidx.at[kk]], add=True)
        plsc.subcore_barrier()
        _copy_out(acc, out_h, c, s, nunits, isem)

    return k(table, edges4)


_BN = 2000


def _xw(x, w1):
    n, din = x.shape
    dh = w1.shape[1]

    def body(x_ref, w_ref, xw_ref):
        xw_ref[...] = jnp.dot(x_ref[...], w_ref[...],
                              preferred_element_type=F32)

    return pl.pallas_call(
        body,
        grid=(n // _BN,),
        in_specs=[
            pl.BlockSpec((_BN, din), lambda i: (i, 0)),
            pl.BlockSpec((din, dh), lambda i: (0, 0)),
        ],
        out_specs=pl.BlockSpec((_BN, dh), lambda i: (i, 0)),
        out_shape=jax.ShapeDtypeStruct((n, dh), F32),
    )(x, w1)


def _pre(xw, degp):
    n, dh = xw.shape

    def body(xw_ref, dp_ref, xws_ref, isd_ref):
        deg = dp_ref[0] + dp_ref[1]
        isd = jnp.where(deg > 0, lax.rsqrt(deg), 0.0)
        isd_col = isd[:, 0:1]
        xws_ref[...] = xw_ref[...] * isd_col
        isd_ref[...] = isd_col

    return pl.pallas_call(
        body,
        grid=(n // _BN,),
        in_specs=[
            pl.BlockSpec((_BN, dh), lambda i: (i, 0)),
            pl.BlockSpec((2, _BN, 16), lambda i: (0, i, 0)),
        ],
        out_specs=[
            pl.BlockSpec((_BN, dh), lambda i: (i, 0)),
            pl.BlockSpec((_BN, 1), lambda i: (i, 0)),
        ],
        out_shape=[
            jax.ShapeDtypeStruct((n, dh), F32),
            jax.ShapeDtypeStruct((n, 1), F32),
        ],
    )(xw, degp)


def _mid(accp, isd, w2):
    _, n, dh = accp.shape
    dout = w2.shape[1]

    def body(a_ref, isd_ref, w_ref, out_ref):
        isd_col = isd_ref[...]
        h = jnp.maximum((a_ref[0] + a_ref[1]) * isd_col, 0.0)
        out_ref[...] = jnp.dot(h, w_ref[...],
                               preferred_element_type=F32) * isd_col

    return pl.pallas_call(
        body,
        grid=(n // _BN,),
        in_specs=[
            pl.BlockSpec((2, _BN, dh), lambda i: (0, i, 0)),
            pl.BlockSpec((_BN, 1), lambda i: (i, 0)),
            pl.BlockSpec((dh, dout), lambda i: (0, 0)),
        ],
        out_specs=pl.BlockSpec((_BN, dout), lambda i: (i, 0)),
        out_shape=jax.ShapeDtypeStruct((n, dout), F32),
    )(accp, isd, w2)


def _post(p, isd):
    _, n, dout = p.shape

    def body(p_ref, isd_ref, out_ref):
        out_ref[...] = (p_ref[0] + p_ref[1]) * isd_ref[...]

    return pl.pallas_call(
        body,
        grid=(n // _BN,),
        in_specs=[
            pl.BlockSpec((2, _BN, dout), lambda i: (0, i, 0)),
            pl.BlockSpec((_BN, 1), lambda i: (i, 0)),
        ],
        out_specs=pl.BlockSpec((_BN, dout), lambda i: (i, 0)),
        out_shape=jax.ShapeDtypeStruct((n, dout), F32),
    )(p, isd)


def kernel(x, edge_index, W1, W2):
    n = x.shape[0]
    e = edge_index.shape[1]
    epw = e // _NW
    edges4 = edge_index.reshape(2, _NW, epw // _C, _C)
    xw = _xw(x, W1)
    degp = _deg_partials(edges4, n)
    xws, isd = _pre(xw, degp)
    accp = _spmm_partials(xws, edges4)
    hws = _mid(accp, isd, W2)
    p2 = _spmm_partials(hws, edges4)
    return _post(p2, isd)


# speedup vs baseline: 1.0534x; 1.0013x over previous
import functools

import jax
import jax.numpy as jnp
from jax import lax
from jax.experimental import pallas as pl
from jax.experimental.pallas import tpu as pltpu
from jax.experimental.pallas import tpu_sc as plsc

F32 = jnp.float32

_NC = 2
_NS = 16
_NW = _NC * _NS
_C = 80
_U = 80


def _sc_mesh():
    return plsc.VectorSubcoreMesh(core_axis_name="c", subcore_axis_name="s")


_SC_PARAMS = pltpu.CompilerParams(use_tc_tiling_on_sc=False)


def _zero_acc(acc, zbuf, s, d, nunits, zsem):
    mtrips = (nunits + _NS - 1) // _NS

    def zinit(i, carry):
        for j in range(d // 16):
            zbuf[i, pl.ds(j * 16, 16)] = jnp.zeros((16,), F32)
        return carry

    lax.fori_loop(0, _U, zinit, 0)

    def zcopy(m, carry):
        t = s + m * _NS

        @pl.when(t < nunits)
        def _():
            pltpu.async_copy(zbuf, acc.at[pl.ds(t * _U, _U)], zsem)

        return carry

    lax.fori_loop(0, mtrips, zcopy, 0)

    def zdrain(m, carry):
        t = s + m * _NS

        @pl.when(t < nunits)
        def _():
            pltpu.make_async_copy(zbuf, acc.at[pl.ds(t * _U, _U)],
                                  zsem).wait()

        return carry

    lax.fori_loop(0, mtrips, zdrain, 0)


def _copy_out(acc, out_h, c, s, nunits, osem):
    mtrips = (nunits + _NS - 1) // _NS

    def ocopy(m, carry):
        t = s + m * _NS

        @pl.when(t < nunits)
        def _():
            pltpu.async_copy(acc.at[pl.ds(t * _U, _U)],
                             out_h.at[c, pl.ds(t * _U, _U)], osem)

        return carry

    lax.fori_loop(0, mtrips, ocopy, 0)

    def odrain(m, carry):
        t = s + m * _NS

        @pl.when(t < nunits)
        def _():
            pltpu.make_async_copy(acc.at[pl.ds(t * _U, _U)],
                                  out_h.at[c, pl.ds(t * _U, _U)], osem).wait()

        return carry

    lax.fori_loop(0, mtrips, odrain, 0)


def _deg_partials(edges4, n):
    _, nw, nch, c_ = edges4.shape
    nunits = n // _U

    @functools.partial(
        pl.kernel,
        out_type=jax.ShapeDtypeStruct((_NC, n, 16), F32),
        mesh=_sc_mesh(),
        compiler_params=_SC_PARAMS,
        scratch_types=[
            pltpu.VMEM_SHARED((n, 16), F32),
            pltpu.VMEM((nch, _C), jnp.int32),
            pltpu.VMEM((_C, 16), F32),
            pltpu.VMEM((_U, 16), F32),
            pltpu.SemaphoreType.DMA,
            pltpu.SemaphoreType.DMA,
            pltpu.SemaphoreType.DMA,
        ],
    )
    def k(edges_h, out_h, acc, uidx, ones_v, zbuf, isem, ssem, zsem):
        c = lax.axis_index("c")
        s = lax.axis_index("s")
        w = c * _NS + s

        pltpu.async_copy(edges_h.at[0, w], uidx, isem)

        def init(i, carry):
            ones_v[i, :] = jnp.full((16,), 1.0, F32)
            return carry

        lax.fori_loop(0, _C, init, 0)
        _zero_acc(acc, zbuf, s, 16, nunits, zsem)
        plsc.subcore_barrier()
        pltpu.make_async_copy(edges_h.at[0, w], uidx, isem).wait()

        lag = 4

        def body(kk, carry):
            pltpu.async_copy(ones_v, acc.at[uidx.at[kk]], ssem, add=True)

            @pl.when(kk >= lag)
            def _():
                pltpu.make_async_copy(ones_v, acc.at[uidx.at[kk]], ssem).wait()

            return carry

        lax.fori_loop(0, nch, body, 0)
        for _ in range(min(lag, nch)):
            pltpu.make_async_copy(ones_v, acc.at[uidx.at[0]], ssem).wait()
        plsc.subcore_barrier()
        _copy_out(acc, out_h, c, s, nunits, isem)

    return k(edges4)


def _spmm_partials(table, edges4):
    n, d = table.shape
    _, nw, nch, c_ = edges4.shape
    nunits = n // _U
    nbuf = 3 if d >= 128 else 8
    ngrp = nch // nbuf
    ntail = nch - ngrp * nbuf

    @functools.partial(
        pl.kernel,
        out_type=jax.ShapeDtypeStruct((_NC, n, d), F32),
        mesh=_sc_mesh(),
        compiler_params=_SC_PARAMS,
        scratch_types=[
            pltpu.VMEM_SHARED((n, d), F32),
            pltpu.VMEM((nch, _C), jnp.int32),
            pltpu.VMEM((nch, _C), jnp.int32),
            [pltpu.VMEM((_C, d), F32)] * nbuf,
            pltpu.SemaphoreType.DMA,
            [pltpu.SemaphoreType.DMA] * nbuf,
            [pltpu.SemaphoreType.DMA] * nbuf,
            pltpu.SemaphoreType.DMA,
        ],
    )
    def k(table_h, edges_h, out_h, acc, vidx, uidx, rows,
          isem, gsem, ssem, zsem):
        c = lax.axis_index("c")
        s = lax.axis_index("s")
        w = c * _NS + s
        src = table_h

        pltpu.async_copy(edges_h.at[1, w], vidx, isem)
        pltpu.async_copy(edges_h.at[0, w], uidx, isem)
        _zero_acc(acc, rows[0], s, d, nunits, zsem)
        pltpu.make_async_copy(edges_h.at[1, w], vidx, isem).wait()
        pltpu.make_async_copy(edges_h.at[0, w], uidx, isem).wait()
        for b in range(nbuf):
            pltpu.async_copy(src.at[vidx.at[b]], rows[b], gsem[b])
        plsc.subcore_barrier()

        def stage(kk, b):
            pltpu.make_async_copy(src.at[vidx.at[kk]], rows[b],
                                  gsem[b]).wait()
            pltpu.async_copy(rows[b], acc.at[uidx.at[kk]], ssem[b], add=True)
            pltpu.make_async_copy(rows[b], acc.at[uidx.at[kk]],
                                  ssem[b]).wait()

            @pl.when(kk + nbuf < nch)
            def _():
                pltpu.async_copy(src.at[vidx.at[kk + nbuf]], rows[b],
                                 gsem[b])

        def body(j, carry):
            for b in range(nbuf):
                stage(nbuf * j + b, b)
            return carry

        lax.fori_loop(0, ngrp, body, 0)
        for t in range(ntail):
            kk = ngrp * nbuf + t
            pltpu.make_async_copy(src.at[vidx.at[kk]], rows[t],
                                  gsem[t]).wait()
            pltpu.sync_copy(rows[t], acc.at[uidx.at[kk]], add=True)
        plsc.subcore_barrier()
        _copy_out(acc, out_h, c, s, nunits, isem)

    return k(table, edges4)


_BN = 2000


def _xw(x, w1):
    n, din = x.shape
    dh = w1.shape[1]

    def body(x_ref, w_ref, xw_ref):
        xw_ref[...] = jnp.dot(x_ref[...], w_ref[...],
                              preferred_element_type=F32)

    return pl.pallas_call(
        body,
        grid=(n // _BN,),
        in_specs=[
            pl.BlockSpec((_BN, din), lambda i: (i, 0)),
            pl.BlockSpec((din, dh), lambda i: (0, 0)),
        ],
        out_specs=pl.BlockSpec((_BN, dh), lambda i: (i, 0)),
        out_shape=jax.ShapeDtypeStruct((n, dh), F32),
    )(x, w1)


def _pre(xw, degp):
    n, dh = xw.shape

    def body(xw_ref, dp_ref, xws_ref, isd_ref):
        deg = dp_ref[0] + dp_ref[1]
        isd = jnp.where(deg > 0, lax.rsqrt(deg), 0.0)
        isd_col = isd[:, 0:1]
        xws_ref[...] = xw_ref[...] * isd_col
        isd_ref[...] = isd_col

    return pl.pallas_call(
        body,
        grid=(n // _BN,),
        in_specs=[
            pl.BlockSpec((_BN, dh), lambda i: (i, 0)),
            pl.BlockSpec((2, _BN, 16), lambda i: (0, i, 0)),
        ],
        out_specs=[
            pl.BlockSpec((_BN, dh), lambda i: (i, 0)),
            pl.BlockSpec((_BN, 1), lambda i: (i, 0)),
        ],
        out_shape=[
            jax.ShapeDtypeStruct((n, dh), F32),
            jax.ShapeDtypeStruct((n, 1), F32),
        ],
    )(xw, degp)


def _mid(accp, isd, w2):
    _, n, dh = accp.shape
    dout = w2.shape[1]

    def body(a_ref, isd_ref, w_ref, out_ref):
        isd_col = isd_ref[...]
        h = jnp.maximum((a_ref[0] + a_ref[1]) * isd_col, 0.0)
        out_ref[...] = jnp.dot(h, w_ref[...],
                               preferred_element_type=F32) * isd_col

    return pl.pallas_call(
        body,
        grid=(n // _BN,),
        in_specs=[
            pl.BlockSpec((2, _BN, dh), lambda i: (0, i, 0)),
            pl.BlockSpec((_BN, 1), lambda i: (i, 0)),
            pl.BlockSpec((dh, dout), lambda i: (0, 0)),
        ],
        out_specs=pl.BlockSpec((_BN, dout), lambda i: (i, 0)),
        out_shape=jax.ShapeDtypeStruct((n, dout), F32),
    )(accp, isd, w2)


def _post(p, isd):
    _, n, dout = p.shape

    def body(p_ref, isd_ref, out_ref):
        out_ref[...] = (p_ref[0] + p_ref[1]) * isd_ref[...]

    return pl.pallas_call(
        body,
        grid=(n // _BN,),
        in_specs=[
            pl.BlockSpec((2, _BN, dout), lambda i: (0, i, 0)),
            pl.BlockSpec((_BN, 1), lambda i: (i, 0)),
        ],
        out_specs=pl.BlockSpec((_BN, dout), lambda i: (i, 0)),
        out_shape=jax.ShapeDtypeStruct((n, dout), F32),
    )(p, isd)


def kernel(x, edge_index, W1, W2):
    n = x.shape[0]
    e = edge_index.shape[1]
    epw = e // _NW
    edges4 = edge_index.reshape(2, _NW, epw // _C, _C)
    xw = _xw(x, W1)
    degp = _deg_partials(edges4, n)
    xws, isd = _pre(xw, degp)
    accp = _spmm_partials(xws, edges4)
    hws = _mid(accp, isd, W2)
    p2 = _spmm_partials(hws, edges4)
    return _post(p2, isd)
